# baseline (speedup 1.0000x reference)
import functools

import jax
import jax.numpy as jnp
from jax import lax
from jax.experimental import pallas as pl
from jax.experimental.pallas import tpu as pltpu
from jax.experimental.pallas import tpu_sc as plsc

NC, NS = 2, 16
NW = NC * NS


def _sc_gather_halves(table, idx, chunk2):
    n = idx.shape[0]
    n2 = n // 2
    e = table.shape[1]
    per_w = n2 // NW
    n_chunks = per_w // chunk2
    mesh = plsc.VectorSubcoreMesh(core_axis_name="c", subcore_axis_name="s")

    @functools.partial(
        pl.kernel,
        mesh=mesh,
        out_type=jax.ShapeDtypeStruct((n2, 2 * e), jnp.float32),
        scratch_types=[
            pltpu.VMEM((chunk2,), jnp.int32),
            pltpu.VMEM((chunk2,), jnp.int32),
            pltpu.VMEM((chunk2, e), jnp.float32),
            pltpu.VMEM((chunk2, e), jnp.float32),
            pltpu.SemaphoreType.DMA,
        ],
        compiler_params=pltpu.CompilerParams(use_tc_tiling_on_sc=False),
    )
    def gather_k(table_hbm, idx_hbm, out_hbm, idxa_v, idxb_v,
                 rows_a, rows_b, sem):
        wid = lax.axis_index("s") * NC + lax.axis_index("c")
        base = wid * per_w

        def body(i, carry):
            off = base + i * chunk2
            pltpu.sync_copy(idx_hbm.at[pl.ds(off, chunk2)], idxa_v)
            pltpu.sync_copy(idx_hbm.at[pl.ds(n2 + off, chunk2)], idxb_v)
            cpa = pltpu.async_copy(table_hbm.at[idxa_v], rows_a, sem)
            cpb = pltpu.async_copy(table_hbm.at[idxb_v], rows_b, sem)
            cpa.wait()
            cpb.wait()
            pltpu.sync_copy(rows_a,
                            out_hbm.at[pl.ds(off, chunk2), pl.ds(0, e)])
            pltpu.sync_copy(rows_b,
                            out_hbm.at[pl.ds(off, chunk2), pl.ds(e, e)])
            return carry

        lax.fori_loop(0, n_chunks, body, 0, unroll=False)

    return gather_k(table, idx)


def _tc_finish_pairs(tok2, pos_a, pos_b, pt_a, pt_b, w2, b128, g128, be128,
                     q, blk2):
    n2 = tok2.shape[0]
    l = pt_a.shape[0]
    h = w2.shape[0] // 2
    grid = n2 // blk2
    pa3 = pos_a.reshape(grid, 1, blk2)
    pb3 = pos_b.reshape(grid, 1, blk2)

    def body(tok_ref, pa_ref, pb_ref, pta_ref, ptb_ref, w_ref, b_ref,
             g_ref, be_ref, q_ref, ln_ref, out_ref):
        x = tok_ref[...]
        pa = pa_ref[0, 0, :]
        pb = pb_ref[0, 0, :]
        oha = (pa[:, None] == lax.broadcasted_iota(jnp.int32, (blk2, l), 1))
        ohb = (pb[:, None] == lax.broadcasted_iota(jnp.int32, (blk2, l), 1))
        pe = jnp.dot(oha.astype(jnp.bfloat16), pta_ref[...],
                     preferred_element_type=jnp.float32)
        pe += jnp.dot(ohb.astype(jnp.bfloat16), ptb_ref[...],
                      preferred_element_type=jnp.float32)
        x = x + pe
        y = jnp.dot(x.astype(jnp.bfloat16), w_ref[...],
                    preferred_element_type=jnp.float32)
        y = y + b_ref[...]
        out_ref[0, :, :] = y[:, :h]
        out_ref[1, :, :] = y[:, h:]
        mean = jnp.dot(y, q_ref[...], preferred_element_type=jnp.float32)
        d = y - mean
        var = jnp.dot(d * d, q_ref[...], preferred_element_type=jnp.float32)
        ln = d * lax.rsqrt(var + 1e-5) * g_ref[...] + be_ref[...]
        ln_ref[0, :, :] = ln[:, :h]
        ln_ref[1, :, :] = ln[:, h:]

    ln, out = pl.pallas_call(
        body,
        grid=(grid,),
        in_specs=[
            pl.BlockSpec((blk2, 128), lambda i: (i, 0)),
            pl.BlockSpec((1, 1, blk2), lambda i: (i, 0, 0)),
            pl.BlockSpec((1, 1, blk2), lambda i: (i, 0, 0)),
            pl.BlockSpec((l, 128), lambda i: (0, 0)),
            pl.BlockSpec((l, 128), lambda i: (0, 0)),
            pl.BlockSpec((128, 128), lambda i: (0, 0)),
            pl.BlockSpec((1, 128), lambda i: (0, 0)),
            pl.BlockSpec((1, 128), lambda i: (0, 0)),
            pl.BlockSpec((1, 128), lambda i: (0, 0)),
            pl.BlockSpec((128, 128), lambda i: (0, 0)),
        ],
        out_specs=[
            pl.BlockSpec((2, blk2, h), lambda i: (0, i, 0)),
            pl.BlockSpec((2, blk2, h), lambda i: (0, i, 0)),
        ],
        out_shape=[
            jax.ShapeDtypeStruct((2, n2, h), jnp.float32),
            jax.ShapeDtypeStruct((2, n2, h), jnp.float32),
        ],
    )(tok2, pa3, pb3, pt_a, pt_b, w2, b128, g128, be128, q)
    return ln, out


def kernel(input_data, pos, token_table, pos_table, W, b, gamma, beta):
    B, S = input_data.shape
    V, E = token_table.shape
    H = W.shape[0]
    L = pos_table.shape[0]
    n = B * S
    n2 = n // 2
    idx_flat = input_data.reshape(n).astype(jnp.int32) * 2
    pos_flat = pos.reshape(n).astype(jnp.int32)
    pos_a = pos_flat[:n2]
    pos_b = pos_flat[n2:]

    t2 = jnp.pad(token_table, ((0, 0), (0, E))).reshape(2 * V, E)
    tok2 = _sc_gather_halves(t2, idx_flat, chunk2=512)

    zero = jnp.zeros((L, E), jnp.float32)
    pt_a = jnp.concatenate([pos_table, zero], axis=1)
    pt_b = jnp.concatenate([zero, pos_table], axis=1)
    wt = W.T
    zw = jnp.zeros((E, H), jnp.float32)
    w2 = jnp.concatenate(
        [jnp.concatenate([wt, zw], axis=1),
         jnp.concatenate([zw, wt], axis=1)], axis=0)
    b128 = jnp.concatenate([b, b]).reshape(1, 2 * H)
    g128 = jnp.concatenate([gamma, gamma]).reshape(1, 2 * H)
    be128 = jnp.concatenate([beta, beta]).reshape(1, 2 * H)
    half = jnp.full((H, H), 1.0 / H, jnp.float32)
    zh = jnp.zeros((H, H), jnp.float32)
    q = jnp.concatenate(
        [jnp.concatenate([half, zh], axis=1),
         jnp.concatenate([zh, half], axis=1)], axis=0)

    ln2, out2 = _tc_finish_pairs(tok2, pos_a, pos_b,
                                 pt_a.astype(jnp.bfloat16),
                                 pt_b.astype(jnp.bfloat16),
                                 w2.astype(jnp.bfloat16),
                                 b128, g128, be128, q, blk2=4096)
    return ln2.reshape(B, S, H), out2.reshape(B, S, H)


# speedup vs baseline: 1.0097x; 1.0097x over previous
import functools

import jax
import jax.numpy as jnp
from jax import lax
from jax.experimental import pallas as pl
from jax.experimental.pallas import tpu as pltpu
from jax.experimental.pallas import tpu_sc as plsc

NC, NS = 2, 16
NW = NC * NS


def _sc_gather_halves(table, idx, chunk2):
    n = idx.shape[0]
    n2 = n // 2
    e = table.shape[1]
    per_w = n2 // NW
    n_chunks = per_w // chunk2
    mesh = plsc.VectorSubcoreMesh(core_axis_name="c", subcore_axis_name="s")

    @functools.partial(
        pl.kernel,
        mesh=mesh,
        out_type=jax.ShapeDtypeStruct((n2, 2 * e), jnp.float32),
        scratch_types=[
            pltpu.VMEM((chunk2,), jnp.int32),
            pltpu.VMEM((chunk2,), jnp.int32),
            pltpu.VMEM((chunk2, e), jnp.float32),
            pltpu.VMEM((chunk2, e), jnp.float32),
            pltpu.SemaphoreType.DMA,
        ],
        compiler_params=pltpu.CompilerParams(use_tc_tiling_on_sc=False),
    )
    def gather_k(table_hbm, idx_hbm, out_hbm, idxa_v, idxb_v,
                 rows_a, rows_b, sem):
        wid = lax.axis_index("s") * NC + lax.axis_index("c")
        base = wid * per_w

        def body(i, carry):
            off = base + i * chunk2
            pltpu.sync_copy(idx_hbm.at[pl.ds(off, chunk2)], idxa_v)
            pltpu.sync_copy(idx_hbm.at[pl.ds(n2 + off, chunk2)], idxb_v)
            cpa = pltpu.async_copy(table_hbm.at[idxa_v], rows_a, sem)
            cpb = pltpu.async_copy(table_hbm.at[idxb_v], rows_b, sem)
            cpa.wait()
            cpb.wait()
            pltpu.sync_copy(rows_a,
                            out_hbm.at[pl.ds(off, chunk2), pl.ds(0, e)])
            pltpu.sync_copy(rows_b,
                            out_hbm.at[pl.ds(off, chunk2), pl.ds(e, e)])
            return carry

        lax.fori_loop(0, n_chunks, body, 0, unroll=False)

    return gather_k(table, idx)


def _tc_finish_pairs(tok2, pos_a, pos_b, pt_a, pt_b, w2, b128, g128, be128,
                     q, blk2):
    n2 = tok2.shape[0]
    l = pt_a.shape[0]
    h = w2.shape[0] // 2
    grid = n2 // blk2
    pa3 = pos_a.reshape(grid, 1, blk2)
    pb3 = pos_b.reshape(grid, 1, blk2)

    def body(tok_ref, pa_ref, pb_ref, pta_ref, ptb_ref, w_ref, b_ref,
             g_ref, be_ref, q_ref, ln_ref, out_ref):
        x = tok_ref[...]
        pa = pa_ref[0, 0, :]
        pb = pb_ref[0, 0, :]
        oha = (pa[:, None] == lax.broadcasted_iota(jnp.int32, (blk2, l), 1))
        ohb = (pb[:, None] == lax.broadcasted_iota(jnp.int32, (blk2, l), 1))
        pe = jnp.dot(oha.astype(jnp.bfloat16), pta_ref[...],
                     preferred_element_type=jnp.float32)
        pe += jnp.dot(ohb.astype(jnp.bfloat16), ptb_ref[...],
                      preferred_element_type=jnp.float32)
        x = x + pe
        y = jnp.dot(x.astype(jnp.bfloat16), w_ref[...],
                    preferred_element_type=jnp.float32)
        y = y + b_ref[...]
        out_ref[0, :, :] = y[:, :h]
        out_ref[1, :, :] = y[:, h:]
        mean = jnp.dot(y, q_ref[...], preferred_element_type=jnp.float32)
        d = y - mean
        var = jnp.dot(d * d, q_ref[...], preferred_element_type=jnp.float32)
        ln = d * lax.rsqrt(var + 1e-5) * g_ref[...] + be_ref[...]
        ln_ref[0, :, :] = ln[:, :h]
        ln_ref[1, :, :] = ln[:, h:]

    ln, out = pl.pallas_call(
        body,
        grid=(grid,),
        in_specs=[
            pl.BlockSpec((blk2, 128), lambda i: (i, 0)),
            pl.BlockSpec((1, 1, blk2), lambda i: (i, 0, 0)),
            pl.BlockSpec((1, 1, blk2), lambda i: (i, 0, 0)),
            pl.BlockSpec((l, 128), lambda i: (0, 0)),
            pl.BlockSpec((l, 128), lambda i: (0, 0)),
            pl.BlockSpec((128, 128), lambda i: (0, 0)),
            pl.BlockSpec((1, 128), lambda i: (0, 0)),
            pl.BlockSpec((1, 128), lambda i: (0, 0)),
            pl.BlockSpec((1, 128), lambda i: (0, 0)),
            pl.BlockSpec((128, 128), lambda i: (0, 0)),
        ],
        out_specs=[
            pl.BlockSpec((2, blk2, h), lambda i: (0, i, 0)),
            pl.BlockSpec((2, blk2, h), lambda i: (0, i, 0)),
        ],
        out_shape=[
            jax.ShapeDtypeStruct((2, n2, h), jnp.float32),
            jax.ShapeDtypeStruct((2, n2, h), jnp.float32),
        ],
    )(tok2, pa3, pb3, pt_a, pt_b, w2, b128, g128, be128, q)
    return ln, out


def kernel(input_data, pos, token_table, pos_table, W, b, gamma, beta):
    B, S = input_data.shape
    V, E = token_table.shape
    H = W.shape[0]
    L = pos_table.shape[0]
    n = B * S
    n2 = n // 2
    idx_flat = input_data.reshape(n).astype(jnp.int32) * 2
    pos_flat = pos.reshape(n).astype(jnp.int32)
    pos_a = pos_flat[:n2]
    pos_b = pos_flat[n2:]

    t2 = jnp.pad(token_table, ((0, 0), (0, E))).reshape(2 * V, E)
    tok2 = _sc_gather_halves(t2, idx_flat, chunk2=512)

    zero = jnp.zeros((L, E), jnp.float32)
    pt_a = jnp.concatenate([pos_table, zero], axis=1)
    pt_b = jnp.concatenate([zero, pos_table], axis=1)
    wt = W.T
    zw = jnp.zeros((E, H), jnp.float32)
    w2 = jnp.concatenate(
        [jnp.concatenate([wt, zw], axis=1),
         jnp.concatenate([zw, wt], axis=1)], axis=0)
    b128 = jnp.concatenate([b, b]).reshape(1, 2 * H)
    g128 = jnp.concatenate([gamma, gamma]).reshape(1, 2 * H)
    be128 = jnp.concatenate([beta, beta]).reshape(1, 2 * H)
    half = jnp.full((H, H), 1.0 / H, jnp.float32)
    zh = jnp.zeros((H, H), jnp.float32)
    q = jnp.concatenate(
        [jnp.concatenate([half, zh], axis=1),
         jnp.concatenate([zh, half], axis=1)], axis=0)

    ln2, out2 = _tc_finish_pairs(tok2, pos_a, pos_b,
                                 pt_a.astype(jnp.bfloat16),
                                 pt_b.astype(jnp.bfloat16),
                                 w2.astype(jnp.bfloat16),
                                 b128, g128, be128, q, blk2=8192)
    return ln2.reshape(B, S, H), out2.reshape(B, S, H)


# speedup vs baseline: 1.0526x; 1.0425x over previous
import functools

import jax
import jax.numpy as jnp
from jax import lax
from jax.experimental import pallas as pl
from jax.experimental.pallas import tpu as pltpu
from jax.experimental.pallas import tpu_sc as plsc

NC, NS = 2, 16
NW = NC * NS


def _sc_gather_halves(table, idx, chunk2):
    n = idx.shape[0]
    n2 = n // 2
    e = table.shape[1]
    per_w = n2 // NW
    n_chunks = per_w // chunk2
    mesh = plsc.VectorSubcoreMesh(core_axis_name="c", subcore_axis_name="s")

    @functools.partial(
        pl.kernel,
        mesh=mesh,
        out_type=jax.ShapeDtypeStruct((n2, 2 * e), jnp.float32),
        scratch_types=[
            pltpu.VMEM((chunk2,), jnp.int32),
            pltpu.VMEM((chunk2,), jnp.int32),
            pltpu.VMEM((chunk2, e), jnp.float32),
            pltpu.VMEM((chunk2, e), jnp.float32),
            pltpu.SemaphoreType.DMA,
        ],
        compiler_params=pltpu.CompilerParams(use_tc_tiling_on_sc=False),
    )
    def gather_k(table_hbm, idx_hbm, out_hbm, idxa_v, idxb_v,
                 rows_a, rows_b, sem):
        wid = lax.axis_index("s") * NC + lax.axis_index("c")
        base = wid * per_w

        def body(i, carry):
            off = base + i * chunk2
            pltpu.sync_copy(idx_hbm.at[pl.ds(off, chunk2)], idxa_v)
            pltpu.sync_copy(idx_hbm.at[pl.ds(n2 + off, chunk2)], idxb_v)
            cpa = pltpu.async_copy(table_hbm.at[idxa_v], rows_a, sem)
            cpb = pltpu.async_copy(table_hbm.at[idxb_v], rows_b, sem)
            cpa.wait()
            cpb.wait()
            pltpu.sync_copy(rows_a,
                            out_hbm.at[pl.ds(off, chunk2), pl.ds(0, e)])
            pltpu.sync_copy(rows_b,
                            out_hbm.at[pl.ds(off, chunk2), pl.ds(e, e)])
            return carry

        lax.fori_loop(0, n_chunks, body, 0, unroll=False)

    return gather_k(table, idx)


def _tc_pad_table(tT, v, e, cb):
    grid = (v + cb - 1) // cb

    def body(t_ref, o_ref):
        xt = t_ref[...]
        y = jnp.transpose(xt)
        o_ref[...] = jnp.concatenate(
            [y, jnp.zeros((cb, e), jnp.float32)], axis=1)

    out = pl.pallas_call(
        body,
        grid=(grid,),
        in_specs=[pl.BlockSpec((e, cb), lambda i: (0, i))],
        out_specs=pl.BlockSpec((cb, 2 * e), lambda i: (i, 0)),
        out_shape=jax.ShapeDtypeStruct((v, 2 * e), jnp.float32),
    )(tT)
    return out.reshape(2 * v, e)


def _tc_finish_pairs(tok2, pos_a, pos_b, pt_a, pt_b, w2, b128, g128, be128,
                     q, blk2):
    n2 = tok2.shape[0]
    l = pt_a.shape[0]
    h = w2.shape[0] // 2
    grid = n2 // blk2
    pa3 = pos_a.reshape(grid, 1, blk2)
    pb3 = pos_b.reshape(grid, 1, blk2)

    def body(tok_ref, pa_ref, pb_ref, pta_ref, ptb_ref, w_ref, b_ref,
             g_ref, be_ref, q_ref, ln_ref, out_ref):
        x = tok_ref[...]
        pa = pa_ref[0, 0, :]
        pb = pb_ref[0, 0, :]
        oha = (pa[:, None] == lax.broadcasted_iota(jnp.int32, (blk2, l), 1))
        ohb = (pb[:, None] == lax.broadcasted_iota(jnp.int32, (blk2, l), 1))
        pe = jnp.dot(oha.astype(jnp.bfloat16), pta_ref[...],
                     preferred_element_type=jnp.float32)
        pe += jnp.dot(ohb.astype(jnp.bfloat16), ptb_ref[...],
                      preferred_element_type=jnp.float32)
        x = x + pe
        y = jnp.dot(x.astype(jnp.bfloat16), w_ref[...],
                    preferred_element_type=jnp.float32)
        y = y + b_ref[...]
        out_ref[0, :, :] = y[:, :h]
        out_ref[1, :, :] = y[:, h:]
        mean = jnp.dot(y, q_ref[...], preferred_element_type=jnp.float32)
        d = y - mean
        var = jnp.dot(d * d, q_ref[...], preferred_element_type=jnp.float32)
        ln = d * lax.rsqrt(var + 1e-5) * g_ref[...] + be_ref[...]
        ln_ref[0, :, :] = ln[:, :h]
        ln_ref[1, :, :] = ln[:, h:]

    ln, out = pl.pallas_call(
        body,
        grid=(grid,),
        in_specs=[
            pl.BlockSpec((blk2, 128), lambda i: (i, 0)),
            pl.BlockSpec((1, 1, blk2), lambda i: (i, 0, 0)),
            pl.BlockSpec((1, 1, blk2), lambda i: (i, 0, 0)),
            pl.BlockSpec((l, 128), lambda i: (0, 0)),
            pl.BlockSpec((l, 128), lambda i: (0, 0)),
            pl.BlockSpec((128, 128), lambda i: (0, 0)),
            pl.BlockSpec((1, 128), lambda i: (0, 0)),
            pl.BlockSpec((1, 128), lambda i: (0, 0)),
            pl.BlockSpec((1, 128), lambda i: (0, 0)),
            pl.BlockSpec((128, 128), lambda i: (0, 0)),
        ],
        out_specs=[
            pl.BlockSpec((2, blk2, h), lambda i: (0, i, 0)),
            pl.BlockSpec((2, blk2, h), lambda i: (0, i, 0)),
        ],
        out_shape=[
            jax.ShapeDtypeStruct((2, n2, h), jnp.float32),
            jax.ShapeDtypeStruct((2, n2, h), jnp.float32),
        ],
    )(tok2, pa3, pb3, pt_a, pt_b, w2, b128, g128, be128, q)
    return ln, out


def kernel(input_data, pos, token_table, pos_table, W, b, gamma, beta):
    B, S = input_data.shape
    V, E = token_table.shape
    H = W.shape[0]
    L = pos_table.shape[0]
    n = B * S
    n2 = n // 2
    idx_flat = input_data.reshape(n).astype(jnp.int32) * 2
    pos_flat = pos.reshape(n).astype(jnp.int32)
    pos_a = pos_flat[:n2]
    pos_b = pos_flat[n2:]

    t2 = _tc_pad_table(token_table.T, V, E, cb=2048)
    tok2 = _sc_gather_halves(t2, idx_flat, chunk2=512)

    zero = jnp.zeros((L, E), jnp.float32)
    pt_a = jnp.concatenate([pos_table, zero], axis=1)
    pt_b = jnp.concatenate([zero, pos_table], axis=1)
    wt = W.T
    zw = jnp.zeros((E, H), jnp.float32)
    w2 = jnp.concatenate(
        [jnp.concatenate([wt, zw], axis=1),
         jnp.concatenate([zw, wt], axis=1)], axis=0)
    b128 = jnp.concatenate([b, b]).reshape(1, 2 * H)
    g128 = jnp.concatenate([gamma, gamma]).reshape(1, 2 * H)
    be128 = jnp.concatenate([beta, beta]).reshape(1, 2 * H)
    half = jnp.full((H, H), 1.0 / H, jnp.float32)
    zh = jnp.zeros((H, H), jnp.float32)
    q = jnp.concatenate(
        [jnp.concatenate([half, zh], axis=1),
         jnp.concatenate([zh, half], axis=1)], axis=0)

    ln2, out2 = _tc_finish_pairs(tok2, pos_a, pos_b,
                                 pt_a.astype(jnp.bfloat16),
                                 pt_b.astype(jnp.bfloat16),
                                 w2.astype(jnp.bfloat16),
                                 b128, g128, be128, q, blk2=8192)
    return ln2.reshape(B, S, H), out2.reshape(B, S, H)


# speedup vs baseline: 1.1621x; 1.1040x over previous
import functools

import jax
import jax.numpy as jnp
from jax import lax
from jax.experimental import pallas as pl
from jax.experimental.pallas import tpu as pltpu
from jax.experimental.pallas import tpu_sc as plsc

NC, NS = 2, 16
NW = NC * NS


def _sc_gather_halves(table, idx, chunk2):
    n = idx.shape[0]
    n2 = n // 2
    e = table.shape[1]
    per_w = n2 // NW
    n_chunks = per_w // chunk2
    mesh = plsc.VectorSubcoreMesh(core_axis_name="c", subcore_axis_name="s")

    @functools.partial(
        pl.kernel,
        mesh=mesh,
        out_type=jax.ShapeDtypeStruct((n2, 2 * e), jnp.float32),
        scratch_types=[
            pltpu.VMEM((chunk2,), jnp.int32),
            pltpu.VMEM((chunk2,), jnp.int32),
            pltpu.VMEM((chunk2, e), jnp.float32),
            pltpu.VMEM((chunk2, e), jnp.float32),
            pltpu.SemaphoreType.DMA,
        ],
        compiler_params=pltpu.CompilerParams(use_tc_tiling_on_sc=False),
    )
    def gather_k(table_hbm, idx_hbm, out_hbm, idxa_v, idxb_v,
                 rows_a, rows_b, sem):
        wid = lax.axis_index("s") * NC + lax.axis_index("c")
        base = wid * per_w

        def body(i, carry):
            off = base + i * chunk2
            pltpu.sync_copy(idx_hbm.at[pl.ds(off, chunk2)], idxa_v)
            pltpu.sync_copy(idx_hbm.at[pl.ds(n2 + off, chunk2)], idxb_v)
            cpa = pltpu.async_copy(table_hbm.at[idxa_v], rows_a, sem)
            cpb = pltpu.async_copy(table_hbm.at[idxb_v], rows_b, sem)
            cpa.wait()
            cpb.wait()
            pltpu.sync_copy(rows_a,
                            out_hbm.at[pl.ds(off, chunk2), pl.ds(0, e)])
            pltpu.sync_copy(rows_b,
                            out_hbm.at[pl.ds(off, chunk2), pl.ds(e, e)])
            return carry

        lax.fori_loop(0, n_chunks, body, 0, unroll=False)

    return gather_k(table, idx)


def _tc_pad_table(tT, v, e, cb):
    grid = (v + cb - 1) // cb

    def body(t_ref, o_ref):
        xt = t_ref[...]
        y = jnp.transpose(xt)
        o_ref[...] = jnp.concatenate(
            [y, jnp.zeros((cb, e), jnp.float32)], axis=1)

    out = pl.pallas_call(
        body,
        grid=(grid,),
        in_specs=[pl.BlockSpec((e, cb), lambda i: (0, i))],
        out_specs=pl.BlockSpec((cb, 2 * e), lambda i: (i, 0)),
        out_shape=jax.ShapeDtypeStruct((v, 2 * e), jnp.float32),
    )(tT)
    return out.reshape(2 * v, e)


def _tc_finish_pairs(tok2, pos_a, pos_b, pt_a, pt_b, w2, b128, g128, be128,
                     q, blk2):
    n2 = tok2.shape[0]
    l = pt_a.shape[0]
    h = w2.shape[0] // 2
    grid = n2 // blk2
    pa3 = pos_a.reshape(grid, 1, blk2)
    pb3 = pos_b.reshape(grid, 1, blk2)

    def body(tok_ref, pa_ref, pb_ref, pta_ref, ptb_ref, w_ref, b_ref,
             g_ref, be_ref, q_ref, ln_ref, out_ref):
        x = tok_ref[...]
        pa = pa_ref[0, 0, :]
        pb = pb_ref[0, 0, :]
        oha = (pa[:, None] == lax.broadcasted_iota(jnp.int32, (blk2, l), 1))
        ohb = (pb[:, None] == lax.broadcasted_iota(jnp.int32, (blk2, l), 1))
        pe = jnp.dot(oha.astype(jnp.bfloat16), pta_ref[...],
                     preferred_element_type=jnp.float32)
        pe += jnp.dot(ohb.astype(jnp.bfloat16), ptb_ref[...],
                      preferred_element_type=jnp.float32)
        x = x + pe
        y = jnp.dot(x.astype(jnp.bfloat16), w_ref[...],
                    preferred_element_type=jnp.float32)
        y = y + b_ref[...]
        out_ref[0, :, :] = y[:, :h]
        out_ref[1, :, :] = y[:, h:]
        mean = jnp.dot(y, q_ref[...], preferred_element_type=jnp.float32)
        d = y - mean
        var = jnp.dot(d * d, q_ref[...], preferred_element_type=jnp.float32)
        ln = d * lax.rsqrt(var + 1e-5) * g_ref[...] + be_ref[...]
        ln_ref[0, :, :] = ln[:, :h]
        ln_ref[1, :, :] = ln[:, h:]

    ln, out = pl.pallas_call(
        body,
        grid=(grid,),
        in_specs=[
            pl.BlockSpec((blk2, 128), lambda i: (i, 0)),
            pl.BlockSpec((1, 1, blk2), lambda i: (i, 0, 0)),
            pl.BlockSpec((1, 1, blk2), lambda i: (i, 0, 0)),
            pl.BlockSpec((l, 128), lambda i: (0, 0)),
            pl.BlockSpec((l, 128), lambda i: (0, 0)),
            pl.BlockSpec((128, 128), lambda i: (0, 0)),
            pl.BlockSpec((1, 128), lambda i: (0, 0)),
            pl.BlockSpec((1, 128), lambda i: (0, 0)),
            pl.BlockSpec((1, 128), lambda i: (0, 0)),
            pl.BlockSpec((128, 128), lambda i: (0, 0)),
        ],
        out_specs=[
            pl.BlockSpec((2, blk2, h), lambda i: (0, i, 0)),
            pl.BlockSpec((2, blk2, h), lambda i: (0, i, 0)),
        ],
        out_shape=[
            jax.ShapeDtypeStruct((2, n2, h), jnp.float32),
            jax.ShapeDtypeStruct((2, n2, h), jnp.float32),
        ],
    )(tok2, pa3, pb3, pt_a, pt_b, w2, b128, g128, be128, q)
    return ln, out


def kernel(input_data, pos, token_table, pos_table, W, b, gamma, beta):
    B, S = input_data.shape
    V, E = token_table.shape
    H = W.shape[0]
    L = pos_table.shape[0]
    n = B * S
    n2 = n // 2
    idx_flat = input_data.reshape(n).astype(jnp.int32) * 2
    pos_flat = pos.reshape(n).astype(jnp.int32)
    pos_a = pos_flat[:n2]
    pos_b = pos_flat[n2:]

    t2 = _tc_pad_table(token_table.T, V, E, cb=4096)
    tok2 = _sc_gather_halves(t2, idx_flat, chunk2=512)

    zero = jnp.zeros((L, E), jnp.float32)
    pt_a = jnp.concatenate([pos_table, zero], axis=1)
    pt_b = jnp.concatenate([zero, pos_table], axis=1)
    wt = W.T
    zw = jnp.zeros((E, H), jnp.float32)
    w2 = jnp.concatenate(
        [jnp.concatenate([wt, zw], axis=1),
         jnp.concatenate([zw, wt], axis=1)], axis=0)
    b128 = jnp.concatenate([b, b]).reshape(1, 2 * H)
    g128 = jnp.concatenate([gamma, gamma]).reshape(1, 2 * H)
    be128 = jnp.concatenate([beta, beta]).reshape(1, 2 * H)
    half = jnp.full((H, H), 1.0 / H, jnp.float32)
    zh = jnp.zeros((H, H), jnp.float32)
    q = jnp.concatenate(
        [jnp.concatenate([half, zh], axis=1),
         jnp.concatenate([zh, half], axis=1)], axis=0)

    ln2, out2 = _tc_finish_pairs(tok2, pos_a, pos_b,
                                 pt_a.astype(jnp.bfloat16),
                                 pt_b.astype(jnp.bfloat16),
                                 w2.astype(jnp.bfloat16),
                                 b128, g128, be128, q, blk2=8192)
    return ln2.reshape(B, S, H), out2.reshape(B, S, H)


# speedup vs baseline: 1.2397x; 1.0668x over previous
import functools

import jax
import jax.numpy as jnp
from jax import lax
from jax.experimental import pallas as pl
from jax.experimental.pallas import tpu as pltpu
from jax.experimental.pallas import tpu_sc as plsc

NC, NS = 2, 16
NW = NC * NS


def _sc_gather_halves(table, idx, chunk2):
    n = idx.shape[0]
    n2 = n // 2
    e = table.shape[1]
    per_w = n2 // NW
    n_chunks = per_w // chunk2
    mesh = plsc.VectorSubcoreMesh(core_axis_name="c", subcore_axis_name="s")

    @functools.partial(
        pl.kernel,
        mesh=mesh,
        out_type=jax.ShapeDtypeStruct((n2, 2 * e), jnp.float32),
        scratch_types=[
            pltpu.VMEM((chunk2,), jnp.int32),
            pltpu.VMEM((chunk2,), jnp.int32),
            pltpu.VMEM((chunk2, e), jnp.float32),
            pltpu.VMEM((chunk2, e), jnp.float32),
            pltpu.SemaphoreType.DMA,
        ],
        compiler_params=pltpu.CompilerParams(use_tc_tiling_on_sc=False),
    )
    def gather_k(table_hbm, idx_hbm, out_hbm, idxa_v, idxb_v,
                 rows_a, rows_b, sem):
        wid = lax.axis_index("s") * NC + lax.axis_index("c")
        base = wid * per_w

        def body(i, carry):
            off = base + i * chunk2
            pltpu.sync_copy(idx_hbm.at[pl.ds(off, chunk2)], idxa_v)
            pltpu.sync_copy(idx_hbm.at[pl.ds(n2 + off, chunk2)], idxb_v)
            cpa = pltpu.async_copy(table_hbm.at[idxa_v], rows_a, sem)
            cpb = pltpu.async_copy(table_hbm.at[idxb_v], rows_b, sem)
            cpa.wait()
            cpb.wait()
            pltpu.sync_copy(rows_a,
                            out_hbm.at[pl.ds(off, chunk2), pl.ds(0, e)])
            pltpu.sync_copy(rows_b,
                            out_hbm.at[pl.ds(off, chunk2), pl.ds(e, e)])
            return carry

        lax.fori_loop(0, n_chunks, body, 0, unroll=False)

    return gather_k(table, idx)


def _tc_pad_table(tT, v, e, cb):
    grid = (v + cb - 1) // cb

    def body(t_ref, o_ref):
        xt = t_ref[...]
        y = jnp.transpose(xt)
        o_ref[...] = jnp.concatenate(
            [y, jnp.zeros((cb, e), jnp.float32)], axis=1)

    out = pl.pallas_call(
        body,
        grid=(grid,),
        in_specs=[pl.BlockSpec((e, cb), lambda i: (0, i))],
        out_specs=pl.BlockSpec((cb, 2 * e), lambda i: (i, 0)),
        out_shape=jax.ShapeDtypeStruct((v, 2 * e), jnp.float32),
    )(tT)
    return out.reshape(2 * v, e)


def _tc_finish_pairs(tok2, pos_a, pos_b, pt_a, pt_b, w2, b128, g128, be128,
                     q, blk2):
    n2 = tok2.shape[0]
    l = pt_a.shape[0]
    h = w2.shape[0] // 2
    grid = n2 // blk2
    pa3 = pos_a.reshape(grid, 1, blk2)
    pb3 = pos_b.reshape(grid, 1, blk2)

    def body(tok_ref, pa_ref, pb_ref, pta_ref, ptb_ref, w_ref, b_ref,
             g_ref, be_ref, q_ref, ln_ref, out_ref):
        x = tok_ref[...]
        pa = pa_ref[0, 0, :]
        pb = pb_ref[0, 0, :]
        oha = (pa[:, None] == lax.broadcasted_iota(jnp.int32, (blk2, l), 1))
        ohb = (pb[:, None] == lax.broadcasted_iota(jnp.int32, (blk2, l), 1))
        pe = jnp.dot(oha.astype(jnp.bfloat16), pta_ref[...],
                     preferred_element_type=jnp.float32)
        pe += jnp.dot(ohb.astype(jnp.bfloat16), ptb_ref[...],
                      preferred_element_type=jnp.float32)
        x = x + pe
        y = jnp.dot(x.astype(jnp.bfloat16), w_ref[...],
                    preferred_element_type=jnp.float32)
        y = y + b_ref[...]
        out_ref[0, :, :] = y[:, :h]
        out_ref[1, :, :] = y[:, h:]
        mean = jnp.dot(y, q_ref[...], preferred_element_type=jnp.float32)
        d = y - mean
        var = jnp.dot(d * d, q_ref[...], preferred_element_type=jnp.float32)
        ln = d * lax.rsqrt(var + 1e-5) * g_ref[...] + be_ref[...]
        ln_ref[0, :, :] = ln[:, :h]
        ln_ref[1, :, :] = ln[:, h:]

    ln, out = pl.pallas_call(
        body,
        grid=(grid,),
        in_specs=[
            pl.BlockSpec((blk2, 128), lambda i: (i, 0)),
            pl.BlockSpec((1, 1, blk2), lambda i: (i, 0, 0)),
            pl.BlockSpec((1, 1, blk2), lambda i: (i, 0, 0)),
            pl.BlockSpec((l, 128), lambda i: (0, 0)),
            pl.BlockSpec((l, 128), lambda i: (0, 0)),
            pl.BlockSpec((128, 128), lambda i: (0, 0)),
            pl.BlockSpec((1, 128), lambda i: (0, 0)),
            pl.BlockSpec((1, 128), lambda i: (0, 0)),
            pl.BlockSpec((1, 128), lambda i: (0, 0)),
            pl.BlockSpec((128, 128), lambda i: (0, 0)),
        ],
        out_specs=[
            pl.BlockSpec((2, blk2, h), lambda i: (0, i, 0)),
            pl.BlockSpec((2, blk2, h), lambda i: (0, i, 0)),
        ],
        out_shape=[
            jax.ShapeDtypeStruct((2, n2, h), jnp.float32),
            jax.ShapeDtypeStruct((2, n2, h), jnp.float32),
        ],
    )(tok2, pa3, pb3, pt_a, pt_b, w2, b128, g128, be128, q)
    return ln, out


def kernel(input_data, pos, token_table, pos_table, W, b, gamma, beta):
    B, S = input_data.shape
    V, E = token_table.shape
    H = W.shape[0]
    L = pos_table.shape[0]
    n = B * S
    n2 = n // 2
    idx_flat = input_data.reshape(n).astype(jnp.int32) * 2
    pos_flat = pos.reshape(n).astype(jnp.int32)
    pos_a = pos_flat[:n2]
    pos_b = pos_flat[n2:]

    t2 = _tc_pad_table(token_table.T, V, E, cb=8192)
    tok2 = _sc_gather_halves(t2, idx_flat, chunk2=512)

    zero = jnp.zeros((L, E), jnp.float32)
    pt_a = jnp.concatenate([pos_table, zero], axis=1)
    pt_b = jnp.concatenate([zero, pos_table], axis=1)
    wt = W.T
    zw = jnp.zeros((E, H), jnp.float32)
    w2 = jnp.concatenate(
        [jnp.concatenate([wt, zw], axis=1),
         jnp.concatenate([zw, wt], axis=1)], axis=0)
    b128 = jnp.concatenate([b, b]).reshape(1, 2 * H)
    g128 = jnp.concatenate([gamma, gamma]).reshape(1, 2 * H)
    be128 = jnp.concatenate([beta, beta]).reshape(1, 2 * H)
    half = jnp.full((H, H), 1.0 / H, jnp.float32)
    zh = jnp.zeros((H, H), jnp.float32)
    q = jnp.concatenate(
        [jnp.concatenate([half, zh], axis=1),
         jnp.concatenate([zh, half], axis=1)], axis=0)

    ln2, out2 = _tc_finish_pairs(tok2, pos_a, pos_b,
                                 pt_a.astype(jnp.bfloat16),
                                 pt_b.astype(jnp.bfloat16),
                                 w2.astype(jnp.bfloat16),
                                 b128, g128, be128, q, blk2=8192)
    return ln2.reshape(B, S, H), out2.reshape(B, S, H)


# speedup vs baseline: 1.2599x; 1.0163x over previous
import functools

import jax
import jax.numpy as jnp
from jax import lax
from jax.experimental import pallas as pl
from jax.experimental.pallas import tpu as pltpu
from jax.experimental.pallas import tpu_sc as plsc

NC, NS = 2, 16
NW = NC * NS


def _sc_gather_halves(table, idx, chunk2):
    n = idx.shape[0]
    n2 = n // 2
    e = table.shape[1]
    per_w = n2 // NW
    n_chunks = per_w // chunk2
    mesh = plsc.VectorSubcoreMesh(core_axis_name="c", subcore_axis_name="s")

    @functools.partial(
        pl.kernel,
        mesh=mesh,
        out_type=jax.ShapeDtypeStruct((n2, 2 * e), jnp.float32),
        scratch_types=[
            pltpu.VMEM((chunk2,), jnp.int32),
            pltpu.VMEM((chunk2,), jnp.int32),
            pltpu.VMEM((chunk2, e), jnp.float32),
            pltpu.VMEM((chunk2, e), jnp.float32),
            pltpu.SemaphoreType.DMA,
        ],
        compiler_params=pltpu.CompilerParams(use_tc_tiling_on_sc=False),
    )
    def gather_k(table_hbm, idx_hbm, out_hbm, idxa_v, idxb_v,
                 rows_a, rows_b, sem):
        wid = lax.axis_index("s") * NC + lax.axis_index("c")
        base = wid * per_w

        def body(i, carry):
            off = base + i * chunk2
            pltpu.sync_copy(idx_hbm.at[pl.ds(off, chunk2)], idxa_v)
            pltpu.sync_copy(idx_hbm.at[pl.ds(n2 + off, chunk2)], idxb_v)
            cpa = pltpu.async_copy(table_hbm.at[idxa_v], rows_a, sem)
            cpb = pltpu.async_copy(table_hbm.at[idxb_v], rows_b, sem)
            cpa.wait()
            cpb.wait()
            pltpu.sync_copy(rows_a,
                            out_hbm.at[pl.ds(off, chunk2), pl.ds(0, e)])
            pltpu.sync_copy(rows_b,
                            out_hbm.at[pl.ds(off, chunk2), pl.ds(e, e)])
            return carry

        lax.fori_loop(0, n_chunks, body, 0, unroll=False)

    return gather_k(table, idx)


def _tc_pad_table(tT, v, e, cb):
    grid = (v + cb - 1) // cb

    def body(t_ref, o_ref):
        xt = t_ref[...]
        y = jnp.transpose(xt)
        o_ref[...] = jnp.concatenate(
            [y, jnp.zeros((cb, e), jnp.float32)], axis=1)

    out = pl.pallas_call(
        body,
        grid=(grid,),
        in_specs=[pl.BlockSpec((e, cb), lambda i: (0, i))],
        out_specs=pl.BlockSpec((cb, 2 * e), lambda i: (i, 0)),
        out_shape=jax.ShapeDtypeStruct((v, 2 * e), jnp.float32),
    )(tT)
    return out.reshape(2 * v, e)


def _tc_finish_pairs(tok2, pos_a, pos_b, pt_a, pt_b, w2, b128, g128, be128,
                     q, blk2):
    n2 = tok2.shape[0]
    l = pt_a.shape[0]
    h = w2.shape[0] // 2
    grid = n2 // blk2
    pa3 = pos_a.reshape(grid, 1, blk2)
    pb3 = pos_b.reshape(grid, 1, blk2)

    def body(tok_ref, pa_ref, pb_ref, pta_ref, ptb_ref, w_ref, b_ref,
             g_ref, be_ref, q_ref, ln_ref, out_ref):
        x = tok_ref[...]
        pa = pa_ref[0, 0, :]
        pb = pb_ref[0, 0, :]
        oha = (pa[:, None] == lax.broadcasted_iota(jnp.int32, (blk2, l), 1))
        ohb = (pb[:, None] == lax.broadcasted_iota(jnp.int32, (blk2, l), 1))
        pe = jnp.dot(oha.astype(jnp.bfloat16), pta_ref[...],
                     preferred_element_type=jnp.float32)
        pe += jnp.dot(ohb.astype(jnp.bfloat16), ptb_ref[...],
                      preferred_element_type=jnp.float32)
        x = x + pe
        y = jnp.dot(x.astype(jnp.bfloat16), w_ref[...],
                    preferred_element_type=jnp.float32)
        y = y + b_ref[...]
        out_ref[0, :, :] = y[:, :h]
        out_ref[1, :, :] = y[:, h:]
        mean = jnp.dot(y, q_ref[...], preferred_element_type=jnp.float32)
        d = y - mean
        var = jnp.dot(d * d, q_ref[...], preferred_element_type=jnp.float32)
        ln = d * lax.rsqrt(var + 1e-5) * g_ref[...] + be_ref[...]
        ln_ref[0, :, :] = ln[:, :h]
        ln_ref[1, :, :] = ln[:, h:]

    ln, out = pl.pallas_call(
        body,
        grid=(grid,),
        in_specs=[
            pl.BlockSpec((blk2, 128), lambda i: (i, 0)),
            pl.BlockSpec((1, 1, blk2), lambda i: (i, 0, 0)),
            pl.BlockSpec((1, 1, blk2), lambda i: (i, 0, 0)),
            pl.BlockSpec((l, 128), lambda i: (0, 0)),
            pl.BlockSpec((l, 128), lambda i: (0, 0)),
            pl.BlockSpec((128, 128), lambda i: (0, 0)),
            pl.BlockSpec((1, 128), lambda i: (0, 0)),
            pl.BlockSpec((1, 128), lambda i: (0, 0)),
            pl.BlockSpec((1, 128), lambda i: (0, 0)),
            pl.BlockSpec((128, 128), lambda i: (0, 0)),
        ],
        out_specs=[
            pl.BlockSpec((2, blk2, h), lambda i: (0, i, 0)),
            pl.BlockSpec((2, blk2, h), lambda i: (0, i, 0)),
        ],
        out_shape=[
            jax.ShapeDtypeStruct((2, n2, h), jnp.float32),
            jax.ShapeDtypeStruct((2, n2, h), jnp.float32),
        ],
    )(tok2, pa3, pb3, pt_a, pt_b, w2, b128, g128, be128, q)
    return ln, out


def kernel(input_data, pos, token_table, pos_table, W, b, gamma, beta):
    B, S = input_data.shape
    V, E = token_table.shape
    H = W.shape[0]
    L = pos_table.shape[0]
    n = B * S
    n2 = n // 2
    idx_flat = input_data.reshape(n).astype(jnp.int32) * 2
    pos_flat = pos.reshape(n).astype(jnp.int32)
    pos_a = pos_flat[:n2]
    pos_b = pos_flat[n2:]

    t2 = _tc_pad_table(token_table.T, V, E, cb=16384)
    tok2 = _sc_gather_halves(t2, idx_flat, chunk2=512)

    zero = jnp.zeros((L, E), jnp.float32)
    pt_a = jnp.concatenate([pos_table, zero], axis=1)
    pt_b = jnp.concatenate([zero, pos_table], axis=1)
    wt = W.T
    zw = jnp.zeros((E, H), jnp.float32)
    w2 = jnp.concatenate(
        [jnp.concatenate([wt, zw], axis=1),
         jnp.concatenate([zw, wt], axis=1)], axis=0)
    b128 = jnp.concatenate([b, b]).reshape(1, 2 * H)
    g128 = jnp.concatenate([gamma, gamma]).reshape(1, 2 * H)
    be128 = jnp.concatenate([beta, beta]).reshape(1, 2 * H)
    half = jnp.full((H, H), 1.0 / H, jnp.float32)
    zh = jnp.zeros((H, H), jnp.float32)
    q = jnp.concatenate(
        [jnp.concatenate([half, zh], axis=1),
         jnp.concatenate([zh, half], axis=1)], axis=0)

    ln2, out2 = _tc_finish_pairs(tok2, pos_a, pos_b,
                                 pt_a.astype(jnp.bfloat16),
                                 pt_b.astype(jnp.bfloat16),
                                 w2.astype(jnp.bfloat16),
                                 b128, g128, be128, q, blk2=8192)
    return ln2.reshape(B, S, H), out2.reshape(B, S, H)
